# baseline (speedup 1.0000x reference)
import functools

import jax
import jax.numpy as jnp
from jax import lax
from jax.experimental import pallas as pl
from jax.experimental.pallas import tpu as pltpu
from jax.experimental.pallas import tpu_sc as plsc

DIM = 32
LANES = 16
NUM_CORES = 2
NUM_SUBCORES = 16
NW = NUM_CORES * NUM_SUBCORES


def kernel(users, items, user_emb, item_emb):
    batch = users.shape[0]
    b_per_w = batch // NW
    mesh = plsc.VectorSubcoreMesh(core_axis_name="c", subcore_axis_name="s")
    cp = pltpu.CompilerParams(
        needs_layout_passes=False, use_tc_tiling_on_sc=False
    )

    @functools.partial(
        pl.kernel,
        compiler_params=cp,
        out_type=jax.ShapeDtypeStruct((batch,), jnp.float32),
        mesh=mesh,
        scratch_types=[
            pltpu.VMEM((b_per_w,), jnp.int32),
            pltpu.VMEM((b_per_w,), jnp.int32),
            pltpu.VMEM((DIM, b_per_w), jnp.float32),
            pltpu.VMEM((DIM, b_per_w), jnp.float32),
            pltpu.VMEM((b_per_w,), jnp.float32),
            pltpu.SemaphoreType.DMA,
        ],
    )
    def sc_kernel(users_hbm, items_hbm, uembt_hbm, vembt_hbm, out_hbm,
                  uidx_v, iidx_v, ug_v, vg_v, out_v, sem):
        wid = lax.axis_index("s") * NUM_CORES + lax.axis_index("c")
        base = wid * b_per_w
        pltpu.sync_copy(users_hbm.at[pl.ds(base, b_per_w)], uidx_v)
        pltpu.sync_copy(items_hbm.at[pl.ds(base, b_per_w)], iidx_v)

        for w in range(4):
            copies = []
            for k in range(w * 8, (w + 1) * 8):
                copies.append(pltpu.async_copy(
                    uembt_hbm.at[k].at[uidx_v], ug_v.at[k], sem))
                copies.append(pltpu.async_copy(
                    vembt_hbm.at[k].at[iidx_v], vg_v.at[k], sem))
            for c in copies:
                c.wait()

        @pl.loop(0, b_per_w // LANES)
        def _(g):
            j0 = g * LANES
            acc = None
            for k in range(DIM):
                u = ug_v[k, pl.ds(j0, LANES)]
                v = vg_v[k, pl.ds(j0, LANES)]
                acc = u * v if acc is None else acc + u * v
            out_v[pl.ds(j0, LANES)] = acc

        pltpu.sync_copy(out_v, out_hbm.at[pl.ds(base, b_per_w)])

    return sc_kernel(users, items, user_emb.T, item_emb.T)
